# baseline (speedup 1.0000x reference)
import jax
import jax.numpy as jnp
from jax.experimental import pallas as pl
from jax.experimental.pallas import tpu as pltpu

_BM = 512
_NCH = 8
_OBUF = 3


def _cdist_manual(
    a_hbm,
    b_hbm,
    out_hbm,
    a_vm,
    b_vm,
    bbf_vm,
    b2_vm,
    out_vm,
    a_sems,
    b_sems,
    o_sems,
):
    m, k = a_vm.shape
    n = b_vm.shape[0]
    nb = m // _BM
    ch = n // _NCH

    a_cps = []
    cp = pltpu.make_async_copy(
        a_hbm.at[pl.ds(0, _BM), :], a_vm.at[pl.ds(0, _BM), :], a_sems.at[0]
    )
    cp.start()
    a_cps.append(cp)
    b_cps = []
    for j in range(_NCH):
        cp = pltpu.make_async_copy(
            b_hbm.at[pl.ds(j * ch, ch), :], b_vm.at[pl.ds(j * ch, ch), :], b_sems.at[j]
        )
        cp.start()
        b_cps.append(cp)
    cp = pltpu.make_async_copy(
        a_hbm.at[pl.ds(_BM, m - _BM), :],
        a_vm.at[pl.ds(_BM, m - _BM), :],
        a_sems.at[1],
    )
    cp.start()
    a_rest_cp = cp

    ones = jnp.ones((1, k), jnp.bfloat16)
    out_cps = [None] * nb

    def band_inputs(i):
        if i == 0:
            a_cps[0].wait()
        elif i == 1:
            a_rest_cp.wait()
        a = a_vm[pl.ds(i * _BM, _BM), :]
        a2 = jnp.sum(a * a, axis=1, keepdims=True)
        a_s = (-2.0 * a).astype(jnp.bfloat16)
        return a2, a_s

    a2, a_s = band_inputs(0)
    for j in range(_NCH):
        b_cps[j].wait()
        sl = pl.ds(j * ch, ch)
        c = b_vm[sl, :].astype(jnp.bfloat16)
        bbf_vm[sl, :] = c
        b2c = jax.lax.dot_general(
            ones, c * c, (((1,), (1,)), ((), ())), preferred_element_type=jnp.float32
        )
        b2_vm[:, sl] = b2c
        mxu = jax.lax.dot_general(
            a_s, c, (((1,), (1,)), ((), ())), preferred_element_type=jnp.float32
        )
        d2 = jnp.maximum((a2 + b2c) + mxu, 1e-30)
        out_vm[0, :, sl] = d2 * jax.lax.rsqrt(d2)
    cp = pltpu.make_async_copy(out_vm.at[0], out_hbm.at[pl.ds(0, _BM), :], o_sems.at[0])
    cp.start()
    out_cps[0] = cp

    for i in range(1, nb):
        buf = i % _OBUF
        if i >= _OBUF:
            out_cps[i - _OBUF].wait()
        a2, a_s = band_inputs(i)
        mxu = jax.lax.dot_general(
            a_s,
            bbf_vm[...],
            (((1,), (1,)), ((), ())),
            preferred_element_type=jnp.float32,
        )
        d2 = jnp.maximum((a2 + b2_vm[...]) + mxu, 1e-30)
        out_vm[buf, :, :] = d2 * jax.lax.rsqrt(d2)
        cp = pltpu.make_async_copy(
            out_vm.at[buf], out_hbm.at[pl.ds(i * _BM, _BM), :], o_sems.at[buf]
        )
        cp.start()
        out_cps[i] = cp

    for i in range(nb - _OBUF, nb):
        out_cps[i].wait()


def kernel(source_mesh, target_mesh, state):
    del state
    m, k = source_mesh.shape
    n = target_mesh.shape[0]
    hbm = pl.BlockSpec(memory_space=pltpu.MemorySpace.HBM)
    return pl.pallas_call(
        _cdist_manual,
        in_specs=[hbm, hbm],
        out_specs=hbm,
        out_shape=jax.ShapeDtypeStruct((m, n), jnp.float32),
        scratch_shapes=[
            pltpu.VMEM((m, k), jnp.float32),
            pltpu.VMEM((n, k), jnp.float32),
            pltpu.VMEM((n, k), jnp.bfloat16),
            pltpu.VMEM((1, n), jnp.float32),
            pltpu.VMEM((_OBUF, _BM, n), jnp.float32),
            pltpu.SemaphoreType.DMA((2,)),
            pltpu.SemaphoreType.DMA((_NCH,)),
            pltpu.SemaphoreType.DMA((_OBUF,)),
        ],
    )(source_mesh, target_mesh)


# speedup vs baseline: 1.0553x; 1.0553x over previous
import jax
import jax.numpy as jnp
from jax.experimental import pallas as pl
from jax.experimental.pallas import tpu as pltpu

_BM = 512
_NCH = 4
_OBUF = 3


def _cdist_manual(
    a_hbm,
    b_hbm,
    out_hbm,
    a_vm,
    b_vm,
    bbf_vm,
    b2_vm,
    out_vm,
    a_sems,
    b_sems,
    o_sems,
):
    m, k = a_vm.shape
    n = b_vm.shape[0]
    nb = m // _BM
    ch = n // _NCH

    a_cps = []
    cp = pltpu.make_async_copy(
        a_hbm.at[pl.ds(0, _BM), :], a_vm.at[pl.ds(0, _BM), :], a_sems.at[0]
    )
    cp.start()
    a_cps.append(cp)
    b_cps = []
    for j in range(_NCH):
        cp = pltpu.make_async_copy(
            b_hbm.at[pl.ds(j * ch, ch), :], b_vm.at[pl.ds(j * ch, ch), :], b_sems.at[j]
        )
        cp.start()
        b_cps.append(cp)
    for i in range(1, nb):
        cp = pltpu.make_async_copy(
            a_hbm.at[pl.ds(i * _BM, _BM), :],
            a_vm.at[pl.ds(i * _BM, _BM), :],
            a_sems.at[i],
        )
        cp.start()
        a_cps.append(cp)

    ones = jnp.ones((1, k), jnp.bfloat16)
    out_cps = [None] * nb

    def band_inputs(i):
        a_cps[i].wait()
        a = a_vm[pl.ds(i * _BM, _BM), :]
        a2 = jnp.sum(a * a, axis=1, keepdims=True)
        a_s = (-2.0 * a).astype(jnp.bfloat16)
        return a2, a_s

    a2, a_s = band_inputs(0)
    for j in range(_NCH):
        b_cps[j].wait()
        sl = pl.ds(j * ch, ch)
        c = b_vm[sl, :].astype(jnp.bfloat16)
        bbf_vm[sl, :] = c
        b2c = jax.lax.dot_general(
            ones, c * c, (((1,), (1,)), ((), ())), preferred_element_type=jnp.float32
        )
        b2_vm[:, sl] = b2c
        mxu = jax.lax.dot_general(
            a_s, c, (((1,), (1,)), ((), ())), preferred_element_type=jnp.float32
        )
        d2 = jnp.maximum((a2 + b2c) + mxu, 1e-30)
        out_vm[0, :, sl] = d2 * jax.lax.rsqrt(d2)
    cp = pltpu.make_async_copy(out_vm.at[0], out_hbm.at[pl.ds(0, _BM), :], o_sems.at[0])
    cp.start(priority=1)
    out_cps[0] = cp

    for i in range(1, nb):
        buf = i % _OBUF
        if i >= _OBUF:
            out_cps[i - _OBUF].wait()
        a2, a_s = band_inputs(i)
        mxu = jax.lax.dot_general(
            a_s,
            bbf_vm[...],
            (((1,), (1,)), ((), ())),
            preferred_element_type=jnp.float32,
        )
        d2 = jnp.maximum((a2 + b2_vm[...]) + mxu, 1e-30)
        out_vm[buf, :, :] = d2 * jax.lax.rsqrt(d2)
        cp = pltpu.make_async_copy(
            out_vm.at[buf], out_hbm.at[pl.ds(i * _BM, _BM), :], o_sems.at[buf]
        )
        cp.start(priority=1)
        out_cps[i] = cp

    for i in range(nb - _OBUF, nb):
        out_cps[i].wait()


def kernel(source_mesh, target_mesh, state):
    del state
    m, k = source_mesh.shape
    n = target_mesh.shape[0]
    hbm = pl.BlockSpec(memory_space=pltpu.MemorySpace.HBM)
    return pl.pallas_call(
        _cdist_manual,
        in_specs=[hbm, hbm],
        out_specs=hbm,
        out_shape=jax.ShapeDtypeStruct((m, n), jnp.float32),
        scratch_shapes=[
            pltpu.VMEM((m, k), jnp.float32),
            pltpu.VMEM((n, k), jnp.float32),
            pltpu.VMEM((n, k), jnp.bfloat16),
            pltpu.VMEM((1, n), jnp.float32),
            pltpu.VMEM((_OBUF, _BM, n), jnp.float32),
            pltpu.SemaphoreType.DMA((m // _BM,)),
            pltpu.SemaphoreType.DMA((_NCH,)),
            pltpu.SemaphoreType.DMA((_OBUF,)),
        ],
    )(source_mesh, target_mesh)


# speedup vs baseline: 1.0639x; 1.0081x over previous
import jax
import jax.numpy as jnp
from jax.experimental import pallas as pl
from jax.experimental.pallas import tpu as pltpu

_BM = 512
_SM = 256
_NCH = 4
_OBUF = 3


def _band_plan(m):
    if m >= 4 * _SM + _BM:
        mid = (m - 4 * _SM) // _BM
        heights = [_SM, _SM] + [_BM] * mid + [_SM, _SM]
    else:
        heights = [_BM] * (m // _BM)
    starts, r = [], 0
    for h in heights:
        starts.append(r)
        r += h
    assert r == m
    return list(zip(starts, heights))


def _cdist_manual(
    a_hbm,
    b_hbm,
    out_hbm,
    a_vm,
    b_vm,
    bbf_vm,
    b2_vm,
    out_vm,
    a_sems,
    b_sems,
    o_sems,
):
    m, k = a_vm.shape
    n = b_vm.shape[0]
    ch = n // _NCH
    bands = _band_plan(m)
    nb = len(bands)

    a_cps = []
    r0, h0 = bands[0]
    cp = pltpu.make_async_copy(
        a_hbm.at[pl.ds(r0, h0), :], a_vm.at[pl.ds(r0, h0), :], a_sems.at[0]
    )
    cp.start()
    a_cps.append(cp)
    b_cps = []
    for j in range(_NCH):
        cp = pltpu.make_async_copy(
            b_hbm.at[pl.ds(j * ch, ch), :], b_vm.at[pl.ds(j * ch, ch), :], b_sems.at[j]
        )
        cp.start()
        b_cps.append(cp)
    for i in range(1, nb):
        ri, hi = bands[i]
        cp = pltpu.make_async_copy(
            a_hbm.at[pl.ds(ri, hi), :], a_vm.at[pl.ds(ri, hi), :], a_sems.at[i]
        )
        cp.start()
        a_cps.append(cp)

    ones = jnp.ones((1, k), jnp.bfloat16)
    out_cps = [None] * nb
    buf_last = [None] * _OBUF

    def band_inputs(i):
        ri, hi = bands[i]
        a_cps[i].wait()
        a = a_vm[pl.ds(ri, hi), :]
        a2 = jnp.sum(a * a, axis=1, keepdims=True)
        a_s = (-2.0 * a).astype(jnp.bfloat16)
        return a2, a_s

    def issue_out(i, buf):
        ri, hi = bands[i]
        cp = pltpu.make_async_copy(
            out_vm.at[buf, pl.ds(0, hi), :],
            out_hbm.at[pl.ds(ri, hi), :],
            o_sems.at[buf],
        )
        cp.start()
        out_cps[i] = cp
        buf_last[buf] = i

    a2, a_s = band_inputs(0)
    for j in range(_NCH):
        b_cps[j].wait()
        sl = pl.ds(j * ch, ch)
        c = b_vm[sl, :].astype(jnp.bfloat16)
        bbf_vm[sl, :] = c
        b2c = jax.lax.dot_general(
            ones, c * c, (((1,), (1,)), ((), ())), preferred_element_type=jnp.float32
        )
        b2_vm[:, sl] = b2c
        mxu = jax.lax.dot_general(
            a_s, c, (((1,), (1,)), ((), ())), preferred_element_type=jnp.float32
        )
        d2 = jnp.maximum((a2 + b2c) + mxu, 1e-30)
        out_vm[0, pl.ds(0, h0), sl] = d2 * jax.lax.rsqrt(d2)
    issue_out(0, 0)

    for i in range(1, nb):
        buf = i % _OBUF
        if buf_last[buf] is not None:
            out_cps[buf_last[buf]].wait()
        a2, a_s = band_inputs(i)
        mxu = jax.lax.dot_general(
            a_s,
            bbf_vm[...],
            (((1,), (1,)), ((), ())),
            preferred_element_type=jnp.float32,
        )
        d2 = jnp.maximum((a2 + b2_vm[...]) + mxu, 1e-30)
        out_vm[buf, pl.ds(0, bands[i][1]), :] = d2 * jax.lax.rsqrt(d2)
        issue_out(i, buf)

    for i in range(nb - _OBUF, nb):
        out_cps[i].wait()


def kernel(source_mesh, target_mesh, state):
    del state
    m, k = source_mesh.shape
    n = target_mesh.shape[0]
    hbm = pl.BlockSpec(memory_space=pltpu.MemorySpace.HBM)
    nb = len(_band_plan(m))
    return pl.pallas_call(
        _cdist_manual,
        in_specs=[hbm, hbm],
        out_specs=hbm,
        out_shape=jax.ShapeDtypeStruct((m, n), jnp.float32),
        scratch_shapes=[
            pltpu.VMEM((m, k), jnp.float32),
            pltpu.VMEM((n, k), jnp.float32),
            pltpu.VMEM((n, k), jnp.bfloat16),
            pltpu.VMEM((1, n), jnp.float32),
            pltpu.VMEM((_OBUF, _BM, n), jnp.float32),
            pltpu.SemaphoreType.DMA((nb,)),
            pltpu.SemaphoreType.DMA((_NCH,)),
            pltpu.SemaphoreType.DMA((_OBUF,)),
        ],
    )(source_mesh, target_mesh)
